# Initial kernel scaffold; baseline (speedup 1.0000x reference)
import jax
import jax.numpy as jnp
from jax.experimental import pallas as pl


def kernel(x, edge_index, batch, W1, b1, g1, be1, W2, b2, g2, be2, W3, b3, g3, be3, fW1, fb1, g4, be4, fW2, fb2, fW3, fb3):
    raise NotImplementedError("write your pallas kernel here")


# speedup vs baseline: 7.5270x; 7.5270x over previous
import functools

import jax
import jax.numpy as jnp
from jax import lax
from jax.experimental import pallas as pl
from jax.experimental.pallas import tpu as pltpu
from jax.experimental.pallas import tpu_sc as plsc

NNODES = 10000
NP = 10240
EDGES = 160000
NGRAPH = 64
RB = 256
NRB = NP // RB
CH = 128
TILES = 16
RPT = NP // TILES
NCH = EDGES // CH
EPSBN = 1e-5
FNN = float(NNODES)


def _sc_mesh():
    return plsc.VectorSubcoreMesh(core_axis_name="c", subcore_axis_name="s",
                                  num_cores=2, num_subcores=TILES)


def _deg_call(dst):
    @functools.partial(
        pl.kernel,
        mesh=_sc_mesh(),
        out_type=jax.ShapeDtypeStruct((NP,), jnp.float32),
        scratch_types=[
            pltpu.VMEM_SHARED((NP,), jnp.float32),
            pltpu.VMEM((RPT,), jnp.float32),
            pltpu.VMEM((CH,), jnp.int32),
            pltpu.VMEM((CH,), jnp.float32),
        ],
    )
    def k(dst_hbm, deg_out, acc, ones_r, dst_v, ones_c):
        c = lax.axis_index("c")
        s = lax.axis_index("s")
        for i in range(RPT // 16):
            ones_r[pl.ds(i * 16, 16)] = jnp.ones((16,), jnp.float32)
        for i in range(CH // 16):
            ones_c[pl.ds(i * 16, 16)] = jnp.ones((16,), jnp.float32)
        r0 = s * RPT

        @pl.when(c == 0)
        def _():
            pltpu.sync_copy(ones_r, acc.at[pl.ds(r0, RPT)])

        plsc.subcore_barrier()

        @pl.when(c == 0)
        def _():
            @pl.loop(s, NCH, step=TILES)
            def _(kk):
                pltpu.sync_copy(dst_hbm.at[pl.ds(kk * CH, CH)], dst_v)
                pltpu.sync_copy(ones_c, acc.at[dst_v], add=True)

        plsc.subcore_barrier()

        @pl.when(c == 0)
        def _():
            pltpu.sync_copy(acc.at[pl.ds(r0, RPT)], deg_out.at[pl.ds(r0, RPT)])

    return k(dst)


def _agg_col_call(src, dst, h):
    @functools.partial(
        pl.kernel,
        mesh=_sc_mesh(),
        out_type=jax.ShapeDtypeStruct((2 * NP, 128), jnp.float32),
        scratch_types=[
            pltpu.VMEM_SHARED((NP, 128), jnp.float32),
            pltpu.VMEM((CH,), jnp.int32),
            pltpu.VMEM((CH,), jnp.int32),
            pltpu.VMEM((CH,), jnp.int32),
            pltpu.VMEM((CH, 128), jnp.float32),
            pltpu.SemaphoreType.DMA,
        ],
    )
    def k(src_hbm, dst_hbm, h_hbm, out_hbm, acc, srcr_v, src_v, dst_v, rows_v, sem):
        c = lax.axis_index("c")
        s = lax.axis_index("s")
        base = c * NP
        r0 = s * RPT
        pltpu.sync_copy(h_hbm.at[pl.ds(base + r0, RPT)], acc.at[pl.ds(r0, RPT)])
        plsc.subcore_barrier()

        @pl.loop(s, NCH, step=TILES)
        def _(kk):
            e0 = kk * CH
            pltpu.sync_copy(src_hbm.at[pl.ds(e0, CH)], srcr_v)
            pltpu.sync_copy(dst_hbm.at[pl.ds(e0, CH)], dst_v)
            for i in range(CH // 16):
                sl = pl.ds(i * 16, 16)
                src_v[sl] = srcr_v[sl] + base
            pltpu.async_copy(h_hbm.at[src_v], rows_v, sem).wait()
            pltpu.sync_copy(rows_v, acc.at[dst_v], add=True)

        plsc.subcore_barrier()
        pltpu.sync_copy(acc.at[pl.ds(r0, RPT)],
                        out_hbm.at[pl.ds(base + r0, RPT)])

    return k(src, dst, h)


def _agg_edge_call(src, dst, h, zrows):
    @functools.partial(
        pl.kernel,
        mesh=_sc_mesh(),
        out_type=jax.ShapeDtypeStruct((2 * NP, 128), jnp.float32),
        scratch_types=[
            pltpu.VMEM_SHARED((NP, 128), jnp.float32),
            pltpu.VMEM((CH,), jnp.int32),
            pltpu.VMEM((CH,), jnp.int32),
            pltpu.VMEM((CH, 128), jnp.float32),
            pltpu.SemaphoreType.DMA,
        ],
    )
    def k(src_hbm, dst_hbm, h_hbm, z_hbm, out_hbm, acc, src_v, dst_v, rows_v, sem):
        c = lax.axis_index("c")
        s = lax.axis_index("s")
        r0 = s * RPT

        @pl.when(c == 0)
        def _():
            pltpu.sync_copy(h_hbm.at[pl.ds(r0, RPT)], acc.at[pl.ds(r0, RPT)])

        @pl.when(c == 1)
        def _():
            pltpu.sync_copy(z_hbm.at[pl.ds(r0, RPT)], acc.at[pl.ds(r0, RPT)])

        plsc.subcore_barrier()
        half = NCH // 2

        @pl.loop(c * half + s, (c + 1) * half, step=TILES)
        def _(kk):
            e0 = kk * CH
            pltpu.sync_copy(src_hbm.at[pl.ds(e0, CH)], src_v)
            pltpu.sync_copy(dst_hbm.at[pl.ds(e0, CH)], dst_v)
            pltpu.async_copy(h_hbm.at[src_v], rows_v, sem).wait()
            pltpu.sync_copy(rows_v, acc.at[dst_v], add=True)

        plsc.subcore_barrier()
        pltpu.sync_copy(acc.at[pl.ds(r0, RPT)],
                        out_hbm.at[pl.ds(c * NP + r0, RPT)])

    return k(src, dst, h, zrows)


def _mm1_call(xp, w1p, deg2):
    def body(x_ref, w_ref, deg_ref, out_ref):
        dis = lax.rsqrt(deg_ref[...])
        h = jnp.dot(x_ref[...].astype(jnp.bfloat16), w_ref[...].astype(jnp.bfloat16),
                    preferred_element_type=jnp.float32)
        out_ref[...] = h * dis

    return pl.pallas_call(
        body,
        grid=(NRB, 2),
        in_specs=[
            pl.BlockSpec((RB, 384), lambda i, j: (i, 0)),
            pl.BlockSpec((384, 128), lambda i, j: (0, j)),
            pl.BlockSpec((RB, 1), lambda i, j: (i, 0)),
        ],
        out_specs=pl.BlockSpec((RB, 128), lambda i, j: (j * NRB + i, 0)),
        out_shape=jax.ShapeDtypeStruct((2 * NP, 128), jnp.float32),
    )(xp, w1p, deg2)


def _relu_rows(a0, a1, dis, b, mode, f2):
    if mode == "col":
        y0 = jnp.maximum(a0 * dis + b[:, :f2], 0.0)
        y1 = jnp.maximum(a1 * dis + b[:, f2:], 0.0)
        return y0, y1
    y = jnp.maximum((a0 + a1) * dis + b, 0.0)
    return y, None


def _bn_affine(st, g, be):
    mu = st[0:1, :] * (1.0 / FNN)
    var = st[1:2, :] * (1.0 / FNN) - mu * mu
    alpha = g * lax.rsqrt(var + EPSBN)
    beta = be - mu * alpha
    return alpha, beta


def _stats_call(a, deg2, b2d, f, mode):
    f2 = f // 2 if mode == "col" else f

    def body(a0_ref, a1_ref, deg_ref, b_ref, out_ref):
        i = pl.program_id(0)

        @pl.when(i == 0)
        def _():
            out_ref[...] = jnp.zeros((8, f), jnp.float32)

        dis = lax.rsqrt(deg_ref[...])
        ridx = i * RB + lax.broadcasted_iota(jnp.int32, (RB, 1), 0)
        mask = (ridx < NNODES).astype(jnp.float32)
        y0, y1 = _relu_rows(a0_ref[...], a1_ref[...], dis, b_ref[...], mode, f2)
        if mode == "col":
            y0 = y0 * mask
            y1 = y1 * mask
            out_ref[0:1, :f2] += jnp.sum(y0, axis=0, keepdims=True)
            out_ref[0:1, f2:] += jnp.sum(y1, axis=0, keepdims=True)
            out_ref[1:2, :f2] += jnp.sum(y0 * y0, axis=0, keepdims=True)
            out_ref[1:2, f2:] += jnp.sum(y1 * y1, axis=0, keepdims=True)
        else:
            y0 = y0 * mask
            out_ref[0:1, :] += jnp.sum(y0, axis=0, keepdims=True)
            out_ref[1:2, :] += jnp.sum(y0 * y0, axis=0, keepdims=True)

    return pl.pallas_call(
        body,
        grid=(NRB,),
        in_specs=[
            pl.BlockSpec((RB, f2), lambda i: (i, 0)),
            pl.BlockSpec((RB, f2), lambda i: (NRB + i, 0)),
            pl.BlockSpec((RB, 1), lambda i: (i, 0)),
            pl.BlockSpec((1, f), lambda i: (0, 0)),
        ],
        out_specs=pl.BlockSpec((8, f), lambda i: (0, 0)),
        out_shape=jax.ShapeDtypeStruct((8, f), jnp.float32),
    )(a, a, deg2, b2d)


def _mm_mid_call(a, deg2, b2d, g2d, be2d, stats, w, fin, mode):
    f2 = fin // 2 if mode == "col" else fin

    def body(a0_ref, a1_ref, deg_ref, b_ref, g_ref, be_ref, st_ref, w_ref,
             out_ref):
        dis = lax.rsqrt(deg_ref[...])
        alpha, beta = _bn_affine(st_ref[...], g_ref[...], be_ref[...])
        y0, y1 = _relu_rows(a0_ref[...], a1_ref[...], dis, b_ref[...], mode, f2)
        w_blk = w_ref[...]
        if mode == "col":
            yn0 = y0 * alpha[:, :f2] + beta[:, :f2]
            yn1 = y1 * alpha[:, f2:] + beta[:, f2:]
            wb = w_blk.astype(jnp.bfloat16)
            h = (jnp.dot(yn0.astype(jnp.bfloat16), wb[:f2, :],
                         preferred_element_type=jnp.float32)
                 + jnp.dot(yn1.astype(jnp.bfloat16), wb[f2:, :],
                           preferred_element_type=jnp.float32))
        else:
            yn = y0 * alpha + beta
            h = jnp.dot(yn.astype(jnp.bfloat16), w_blk.astype(jnp.bfloat16),
                        preferred_element_type=jnp.float32)
        out_ref[...] = h * dis

    return pl.pallas_call(
        body,
        grid=(NRB,),
        in_specs=[
            pl.BlockSpec((RB, f2), lambda i: (i, 0)),
            pl.BlockSpec((RB, f2), lambda i: (NRB + i, 0)),
            pl.BlockSpec((RB, 1), lambda i: (i, 0)),
            pl.BlockSpec((1, fin), lambda i: (0, 0)),
            pl.BlockSpec((1, fin), lambda i: (0, 0)),
            pl.BlockSpec((1, fin), lambda i: (0, 0)),
            pl.BlockSpec((8, fin), lambda i: (0, 0)),
            pl.BlockSpec((fin, 128), lambda i: (0, 0)),
        ],
        out_specs=pl.BlockSpec((RB, 128), lambda i: (i, 0)),
        out_shape=jax.ShapeDtypeStruct((NP, 128), jnp.float32),
    )(a, a, deg2, b2d, g2d, be2d, stats, w)


def _pool_call(a, deg2, b2d, g2d, be2d, stats, batch2d):
    def body(a0_ref, a1_ref, deg_ref, b_ref, g_ref, be_ref, st_ref, bat_ref,
             out_ref):
        i = pl.program_id(0)

        @pl.when(i == 0)
        def _():
            out_ref[...] = jnp.zeros((NGRAPH, 128), jnp.float32)

        dis = lax.rsqrt(deg_ref[...])
        alpha, beta = _bn_affine(st_ref[...], g_ref[...], be_ref[...])
        y, _ = _relu_rows(a0_ref[...], a1_ref[...], dis, b_ref[...], "sum", 128)
        yn = y * alpha + beta
        gid = lax.broadcasted_iota(jnp.int32, (RB, NGRAPH), 1)
        onehot = (bat_ref[...] == gid).astype(jnp.float32)
        dn = (((0,), (0,)), ((), ()))
        out_ref[...] += lax.dot_general(onehot, yn, dn,
                                        preferred_element_type=jnp.float32,
                                        precision=lax.Precision.HIGHEST)

    return pl.pallas_call(
        body,
        grid=(NRB,),
        in_specs=[
            pl.BlockSpec((RB, 128), lambda i: (i, 0)),
            pl.BlockSpec((RB, 128), lambda i: (NRB + i, 0)),
            pl.BlockSpec((RB, 1), lambda i: (i, 0)),
            pl.BlockSpec((1, 128), lambda i: (0, 0)),
            pl.BlockSpec((1, 128), lambda i: (0, 0)),
            pl.BlockSpec((1, 128), lambda i: (0, 0)),
            pl.BlockSpec((8, 128), lambda i: (0, 0)),
            pl.BlockSpec((RB, 1), lambda i: (i, 0)),
        ],
        out_specs=pl.BlockSpec((NGRAPH, 128), lambda i: (0, 0)),
        out_shape=jax.ShapeDtypeStruct((NGRAPH, 128), jnp.float32),
    )(a, a, deg2, b2d, g2d, be2d, stats, batch2d)


def _head_call(p, fw1, fb1, g4, be4, fw2, fb2, fw3p, fb3p):
    def body(p_ref, w1_ref, b1_ref, g_ref, be_ref, w2_ref, b2_ref, w3_ref,
             b3_ref, out_ref):
        q = jnp.maximum(
            jnp.dot(p_ref[...].astype(jnp.bfloat16), w1_ref[...].astype(jnp.bfloat16),
                    preferred_element_type=jnp.float32) + b1_ref[...], 0.0)
        mu = jnp.mean(q, axis=0, keepdims=True)
        var = jnp.mean(q * q, axis=0, keepdims=True) - mu * mu
        qn = (q - mu) * lax.rsqrt(var + EPSBN) * g_ref[...] + be_ref[...]
        r = jnp.maximum(
            jnp.dot(qn.astype(jnp.bfloat16), w2_ref[...].astype(jnp.bfloat16),
                    preferred_element_type=jnp.float32) + b2_ref[...], 0.0)
        t = jnp.dot(r.astype(jnp.bfloat16), w3_ref[...].astype(jnp.bfloat16),
                    preferred_element_type=jnp.float32) + b3_ref[...]
        out_ref[...] = jnp.maximum(t, 0.0)

    return pl.pallas_call(
        body,
        out_shape=jax.ShapeDtypeStruct((NGRAPH, 8), jnp.float32),
    )(p, fw1, fb1, g4, be4, fw2, fb2, fw3p, fb3p)


def kernel(x, edge_index, batch, W1, b1, g1, be1, W2, b2, g2, be2, W3, b3,
           g3, be3, fW1, fb1, g4, be4, fW2, fb2, fW3, fb3):
    src = edge_index[0]
    dst = edge_index[1]
    xp = jnp.zeros((NP, 384), jnp.float32).at[:NNODES, :373].set(x)
    w1p = jnp.zeros((384, 256), jnp.float32).at[:373, :].set(W1)
    batch2d = jnp.full((NP, 1), NGRAPH, jnp.int32).at[:NNODES, 0].set(batch)
    zrows = jnp.zeros((NP, 128), jnp.float32)

    deg = _deg_call(dst)
    deg2 = deg.reshape(NP, 1)

    h1 = _mm1_call(xp, w1p, deg2)
    a1 = _agg_col_call(src, dst, h1)
    s1 = _stats_call(a1, deg2, b1.reshape(1, 256), 256, "col")
    h2 = _mm_mid_call(a1, deg2, b1.reshape(1, 256), g1.reshape(1, 256),
                      be1.reshape(1, 256), s1, W2, 256, "col")
    a2 = _agg_edge_call(src, dst, h2, zrows)
    s2 = _stats_call(a2, deg2, b2.reshape(1, 128), 128, "sum")
    h3 = _mm_mid_call(a2, deg2, b2.reshape(1, 128), g2.reshape(1, 128),
                      be2.reshape(1, 128), s2, W3, 128, "sum")
    a3 = _agg_edge_call(src, dst, h3, zrows)
    s3 = _stats_call(a3, deg2, b3.reshape(1, 128), 128, "sum")
    p = _pool_call(a3, deg2, b3.reshape(1, 128), g3.reshape(1, 128),
                   be3.reshape(1, 128), s3, batch2d)

    fw3p = jnp.zeros((64, 8), jnp.float32).at[:, :1].set(fW3)
    fb3p = jnp.zeros((1, 8), jnp.float32).at[0, 0].set(fb3[0])
    out = _head_call(p, fW1, fb1.reshape(1, 64), g4.reshape(1, 64),
                     be4.reshape(1, 64), fW2, fb2.reshape(1, 64), fw3p, fb3p)
    return out[:, 0]


# speedup vs baseline: 12.6289x; 1.6778x over previous
import functools

import jax
import jax.numpy as jnp
from jax import lax
from jax.experimental import pallas as pl
from jax.experimental.pallas import tpu as pltpu
from jax.experimental.pallas import tpu_sc as plsc

NNODES = 10000
NP = 10240
EDGES = 160000
NGRAPH = 64
RB = 256
NRB = NP // RB
CH = 128
TILES = 16
RPT = NP // TILES
NCH = EDGES // CH
GC = 10
GE = GC * CH
NGRP = EDGES // GE
EPSBN = 1e-5
FNN = float(NNODES)


def _sc_mesh():
    return plsc.VectorSubcoreMesh(core_axis_name="c", subcore_axis_name="s",
                                  num_cores=2, num_subcores=TILES)


def _deg_call(dst):
    @functools.partial(
        pl.kernel,
        mesh=_sc_mesh(),
        out_type=jax.ShapeDtypeStruct((NP,), jnp.float32),
        scratch_types=[
            pltpu.VMEM_SHARED((NP,), jnp.float32),
            pltpu.VMEM((RPT,), jnp.float32),
            pltpu.VMEM((GE,), jnp.int32),
            pltpu.VMEM((CH,), jnp.int32),
            pltpu.VMEM((CH,), jnp.int32),
            pltpu.VMEM((CH,), jnp.float32),
            pltpu.SemaphoreType.DMA,
        ],
    )
    def k(dst_hbm, deg_out, acc, ones_r, dstg_v, dsta_v, dstb_v, ones_c, sem):
        c = lax.axis_index("c")
        s = lax.axis_index("s")
        for i in range(RPT // 16):
            ones_r[pl.ds(i * 16, 16)] = jnp.ones((16,), jnp.float32)
        for i in range(CH // 16):
            ones_c[pl.ds(i * 16, 16)] = jnp.ones((16,), jnp.float32)
        r0 = s * RPT
        dstb = [dsta_v, dstb_v]

        @pl.when(c == 0)
        def _():
            pltpu.sync_copy(ones_r, acc.at[pl.ds(r0, RPT)])

        plsc.subcore_barrier()

        @pl.when(c == 0)
        def _():
            @pl.loop(s, NGRP, step=TILES)
            def _(g):
                pltpu.sync_copy(dst_hbm.at[pl.ds(g * GE, GE)], dstg_v)
                descs = [None] * GC
                for j in range(GC):
                    nb = j % 2
                    if j >= 2:
                        descs[j - 2].wait()
                    for i in range(CH // 16):
                        sl = pl.ds(i * 16, 16)
                        dstb[nb][sl] = dstg_v[pl.ds(j * CH + i * 16, 16)]
                    descs[j] = pltpu.async_copy(ones_c, acc.at[dstb[nb]],
                                                sem, add=True)
                descs[GC - 2].wait()
                descs[GC - 1].wait()

        plsc.subcore_barrier()

        @pl.when(c == 0)
        def _():
            pltpu.sync_copy(acc.at[pl.ds(r0, RPT)], deg_out.at[pl.ds(r0, RPT)])

    return k(dst)


def _agg_col_call(src, dst, h):
    @functools.partial(
        pl.kernel,
        mesh=_sc_mesh(),
        out_type=jax.ShapeDtypeStruct((2 * NP, 128), jnp.float32),
        scratch_types=[
            pltpu.VMEM_SHARED((NP, 128), jnp.float32),
            pltpu.VMEM((GE,), jnp.int32),
            pltpu.VMEM((GE,), jnp.int32),
            pltpu.VMEM((CH,), jnp.int32),
            pltpu.VMEM((CH,), jnp.int32),
            pltpu.VMEM((CH,), jnp.int32),
            pltpu.VMEM((CH, 128), jnp.float32),
            pltpu.VMEM((CH, 128), jnp.float32),
            pltpu.SemaphoreType.DMA,
        ],
    )
    def k(src_hbm, dst_hbm, h_hbm, out_hbm, acc, srcg_v, dstg_v, sca_v, scb_v,
          dstc_v, rowa_v, rowb_v, sem):
        c = lax.axis_index("c")
        s = lax.axis_index("s")
        base = c * NP
        r0 = s * RPT
        pltpu.sync_copy(h_hbm.at[pl.ds(base + r0, RPT)], acc.at[pl.ds(r0, RPT)])
        plsc.subcore_barrier()
        srcb = [sca_v, scb_v]
        rowb = [rowa_v, rowb_v]

        @pl.loop(s, NGRP, step=TILES)
        def _(g):
            e0 = g * GE
            pltpu.sync_copy(src_hbm.at[pl.ds(e0, GE)], srcg_v)
            pltpu.sync_copy(dst_hbm.at[pl.ds(e0, GE)], dstg_v)
            for i in range(CH // 16):
                sl = pl.ds(i * 16, 16)
                srcb[0][sl] = srcg_v[sl] + base
            cp = pltpu.async_copy(h_hbm.at[srcb[0]], rowb[0], sem)
            for j in range(GC):
                if j + 1 < GC:
                    nb = (j + 1) % 2
                    for i in range(CH // 16):
                        sl = pl.ds(i * 16, 16)
                        srcb[nb][sl] = srcg_v[pl.ds((j + 1) * CH + i * 16, 16)] + base
                    cpn = pltpu.async_copy(h_hbm.at[srcb[nb]], rowb[nb], sem)
                for i in range(CH // 16):
                    sl = pl.ds(i * 16, 16)
                    dstc_v[sl] = dstg_v[pl.ds(j * CH + i * 16, 16)]
                cp.wait()
                pltpu.sync_copy(rowb[j % 2], acc.at[dstc_v], add=True)
                if j + 1 < GC:
                    cp = cpn

        plsc.subcore_barrier()
        pltpu.sync_copy(acc.at[pl.ds(r0, RPT)],
                        out_hbm.at[pl.ds(base + r0, RPT)])

    return k(src, dst, h)


def _agg_edge_call(src, dst, h, zrows):
    @functools.partial(
        pl.kernel,
        mesh=_sc_mesh(),
        out_type=jax.ShapeDtypeStruct((2 * NP, 128), jnp.float32),
        scratch_types=[
            pltpu.VMEM_SHARED((NP, 128), jnp.float32),
            pltpu.VMEM((GE,), jnp.int32),
            pltpu.VMEM((GE,), jnp.int32),
            pltpu.VMEM((CH,), jnp.int32),
            pltpu.VMEM((CH, 128), jnp.float32),
            pltpu.VMEM((CH, 128), jnp.float32),
            pltpu.SemaphoreType.DMA,
        ],
    )
    def k(src_hbm, dst_hbm, h_hbm, z_hbm, out_hbm, acc, srcg_v, dstg_v,
          dstc_v, rowa_v, rowb_v, sem):
        c = lax.axis_index("c")
        s = lax.axis_index("s")
        r0 = s * RPT

        @pl.when(c == 0)
        def _():
            pltpu.sync_copy(h_hbm.at[pl.ds(r0, RPT)], acc.at[pl.ds(r0, RPT)])

        @pl.when(c == 1)
        def _():
            pltpu.sync_copy(z_hbm.at[pl.ds(r0, RPT)], acc.at[pl.ds(r0, RPT)])

        plsc.subcore_barrier()
        rowb = [rowa_v, rowb_v]
        half = NGRP // 2

        @pl.loop(c * half + s, half + c * (NGRP - half), step=TILES)
        def _(g):
            e0 = g * GE
            pltpu.sync_copy(src_hbm.at[pl.ds(e0, GE)], srcg_v)
            pltpu.sync_copy(dst_hbm.at[pl.ds(e0, GE)], dstg_v)
            cp = pltpu.async_copy(h_hbm.at[srcg_v.at[pl.ds(0, CH)]], rowb[0], sem)
            for j in range(GC):
                if j + 1 < GC:
                    cpn = pltpu.async_copy(
                        h_hbm.at[srcg_v.at[pl.ds((j + 1) * CH, CH)]],
                        rowb[(j + 1) % 2], sem)
                for i in range(CH // 16):
                    sl = pl.ds(i * 16, 16)
                    dstc_v[sl] = dstg_v[pl.ds(j * CH + i * 16, 16)]
                cp.wait()
                pltpu.sync_copy(rowb[j % 2], acc.at[dstc_v], add=True)
                if j + 1 < GC:
                    cp = cpn

        plsc.subcore_barrier()
        pltpu.sync_copy(acc.at[pl.ds(r0, RPT)],
                        out_hbm.at[pl.ds(c * NP + r0, RPT)])

    return k(src, dst, h, zrows)


def _mm1_call(xp, w1p, deg2):
    def body(x_ref, w_ref, deg_ref, out_ref):
        i = pl.program_id(0)
        dis = lax.rsqrt(deg_ref[...])
        h = jnp.dot(x_ref[...].astype(jnp.bfloat16), w_ref[...].astype(jnp.bfloat16),
                    preferred_element_type=jnp.float32)
        ridx = i * RB + lax.broadcasted_iota(jnp.int32, (RB, 1), 0)
        out_ref[...] = jnp.where(ridx < NNODES, h * dis, 0.0)

    return pl.pallas_call(
        body,
        grid=(NRB, 2),
        in_specs=[
            pl.BlockSpec((RB, 373), lambda i, j: (i, 0)),
            pl.BlockSpec((373, 128), lambda i, j: (0, j)),
            pl.BlockSpec((RB, 1), lambda i, j: (i, 0)),
        ],
        out_specs=pl.BlockSpec((RB, 128), lambda i, j: (j * NRB + i, 0)),
        out_shape=jax.ShapeDtypeStruct((2 * NP, 128), jnp.float32),
    )(xp, w1p, deg2)


def _relu_rows(a0, a1, dis, b, mode, f2):
    if mode == "col":
        y0 = jnp.maximum(a0 * dis + b[:, :f2], 0.0)
        y1 = jnp.maximum(a1 * dis + b[:, f2:], 0.0)
        return y0, y1
    y = jnp.maximum((a0 + a1) * dis + b, 0.0)
    return y, None


def _bn_affine(st, g, be):
    mu = st[0:1, :] * (1.0 / FNN)
    var = st[1:2, :] * (1.0 / FNN) - mu * mu
    alpha = g * lax.rsqrt(var + EPSBN)
    beta = be - mu * alpha
    return alpha, beta


def _stats_call(a, deg2, b2d, f, mode):
    f2 = f // 2 if mode == "col" else f

    def body(a0_ref, a1_ref, deg_ref, b_ref, out_ref):
        i = pl.program_id(0)

        @pl.when(i == 0)
        def _():
            out_ref[...] = jnp.zeros((8, f), jnp.float32)

        dis = lax.rsqrt(deg_ref[...])
        ridx = i * RB + lax.broadcasted_iota(jnp.int32, (RB, 1), 0)
        mask = (ridx < NNODES).astype(jnp.float32)
        y0, y1 = _relu_rows(a0_ref[...], a1_ref[...], dis, b_ref[...], mode, f2)
        if mode == "col":
            y0 = y0 * mask
            y1 = y1 * mask
            out_ref[0:1, :f2] += jnp.sum(y0, axis=0, keepdims=True)
            out_ref[0:1, f2:] += jnp.sum(y1, axis=0, keepdims=True)
            out_ref[1:2, :f2] += jnp.sum(y0 * y0, axis=0, keepdims=True)
            out_ref[1:2, f2:] += jnp.sum(y1 * y1, axis=0, keepdims=True)
        else:
            y0 = y0 * mask
            out_ref[0:1, :] += jnp.sum(y0, axis=0, keepdims=True)
            out_ref[1:2, :] += jnp.sum(y0 * y0, axis=0, keepdims=True)

    return pl.pallas_call(
        body,
        grid=(NRB,),
        in_specs=[
            pl.BlockSpec((RB, f2), lambda i: (i, 0)),
            pl.BlockSpec((RB, f2), lambda i: (NRB + i, 0)),
            pl.BlockSpec((RB, 1), lambda i: (i, 0)),
            pl.BlockSpec((1, f), lambda i: (0, 0)),
        ],
        out_specs=pl.BlockSpec((8, f), lambda i: (0, 0)),
        out_shape=jax.ShapeDtypeStruct((8, f), jnp.float32),
    )(a, a, deg2, b2d)


def _mm_mid_call(a, deg2, b2d, g2d, be2d, stats, w, fin, mode):
    f2 = fin // 2 if mode == "col" else fin

    def body(a0_ref, a1_ref, deg_ref, b_ref, g_ref, be_ref, st_ref, w_ref,
             out_ref):
        dis = lax.rsqrt(deg_ref[...])
        alpha, beta = _bn_affine(st_ref[...], g_ref[...], be_ref[...])
        y0, y1 = _relu_rows(a0_ref[...], a1_ref[...], dis, b_ref[...], mode, f2)
        w_blk = w_ref[...]
        if mode == "col":
            yn0 = y0 * alpha[:, :f2] + beta[:, :f2]
            yn1 = y1 * alpha[:, f2:] + beta[:, f2:]
            wb = w_blk.astype(jnp.bfloat16)
            h = (jnp.dot(yn0.astype(jnp.bfloat16), wb[:f2, :],
                         preferred_element_type=jnp.float32)
                 + jnp.dot(yn1.astype(jnp.bfloat16), wb[f2:, :],
                           preferred_element_type=jnp.float32))
        else:
            yn = y0 * alpha + beta
            h = jnp.dot(yn.astype(jnp.bfloat16), w_blk.astype(jnp.bfloat16),
                        preferred_element_type=jnp.float32)
        out_ref[...] = h * dis

    return pl.pallas_call(
        body,
        grid=(NRB,),
        in_specs=[
            pl.BlockSpec((RB, f2), lambda i: (i, 0)),
            pl.BlockSpec((RB, f2), lambda i: (NRB + i, 0)),
            pl.BlockSpec((RB, 1), lambda i: (i, 0)),
            pl.BlockSpec((1, fin), lambda i: (0, 0)),
            pl.BlockSpec((1, fin), lambda i: (0, 0)),
            pl.BlockSpec((1, fin), lambda i: (0, 0)),
            pl.BlockSpec((8, fin), lambda i: (0, 0)),
            pl.BlockSpec((fin, 128), lambda i: (0, 0)),
        ],
        out_specs=pl.BlockSpec((RB, 128), lambda i: (i, 0)),
        out_shape=jax.ShapeDtypeStruct((NP, 128), jnp.float32),
    )(a, a, deg2, b2d, g2d, be2d, stats, w)


def _pool_call(a, deg2, b2d, g2d, be2d, stats, batch2d):
    def body(a0_ref, a1_ref, deg_ref, b_ref, g_ref, be_ref, st_ref, bat_ref,
             out_ref):
        i = pl.program_id(0)

        @pl.when(i == 0)
        def _():
            out_ref[...] = jnp.zeros((NGRAPH, 128), jnp.float32)

        dis = lax.rsqrt(deg_ref[...])
        alpha, beta = _bn_affine(st_ref[...], g_ref[...], be_ref[...])
        y, _ = _relu_rows(a0_ref[...], a1_ref[...], dis, b_ref[...], "sum", 128)
        yn = y * alpha + beta
        gid = lax.broadcasted_iota(jnp.int32, (RB, NGRAPH), 1)
        onehot = (bat_ref[...] == gid).astype(jnp.float32)
        dn = (((0,), (0,)), ((), ()))
        out_ref[...] += lax.dot_general(onehot, yn, dn,
                                        preferred_element_type=jnp.float32,
                                        precision=lax.Precision.HIGHEST)

    return pl.pallas_call(
        body,
        grid=(NRB,),
        in_specs=[
            pl.BlockSpec((RB, 128), lambda i: (i, 0)),
            pl.BlockSpec((RB, 128), lambda i: (NRB + i, 0)),
            pl.BlockSpec((RB, 1), lambda i: (i, 0)),
            pl.BlockSpec((1, 128), lambda i: (0, 0)),
            pl.BlockSpec((1, 128), lambda i: (0, 0)),
            pl.BlockSpec((1, 128), lambda i: (0, 0)),
            pl.BlockSpec((8, 128), lambda i: (0, 0)),
            pl.BlockSpec((RB, 1), lambda i: (i, 0)),
        ],
        out_specs=pl.BlockSpec((NGRAPH, 128), lambda i: (0, 0)),
        out_shape=jax.ShapeDtypeStruct((NGRAPH, 128), jnp.float32),
    )(a, a, deg2, b2d, g2d, be2d, stats, batch2d)


def _head_call(p, fw1, fb1, g4, be4, fw2, fb2, fw3p, fb3p):
    def body(p_ref, w1_ref, b1_ref, g_ref, be_ref, w2_ref, b2_ref, w3_ref,
             b3_ref, out_ref):
        q = jnp.maximum(
            jnp.dot(p_ref[...].astype(jnp.bfloat16), w1_ref[...].astype(jnp.bfloat16),
                    preferred_element_type=jnp.float32) + b1_ref[...], 0.0)
        mu = jnp.mean(q, axis=0, keepdims=True)
        var = jnp.mean(q * q, axis=0, keepdims=True) - mu * mu
        qn = (q - mu) * lax.rsqrt(var + EPSBN) * g_ref[...] + be_ref[...]
        r = jnp.maximum(
            jnp.dot(qn.astype(jnp.bfloat16), w2_ref[...].astype(jnp.bfloat16),
                    preferred_element_type=jnp.float32) + b2_ref[...], 0.0)
        t = jnp.dot(r.astype(jnp.bfloat16), w3_ref[...].astype(jnp.bfloat16),
                    preferred_element_type=jnp.float32) + b3_ref[...]
        out_ref[...] = jnp.maximum(t, 0.0)

    return pl.pallas_call(
        body,
        out_shape=jax.ShapeDtypeStruct((NGRAPH, 8), jnp.float32),
    )(p, fw1, fb1, g4, be4, fw2, fb2, fw3p, fb3p)


def kernel(x, edge_index, batch, W1, b1, g1, be1, W2, b2, g2, be2, W3, b3,
           g3, be3, fW1, fb1, g4, be4, fW2, fb2, fW3, fb3):
    src = edge_index[0]
    dst = edge_index[1]
    batch2d = jnp.full((NP, 1), NGRAPH, jnp.int32).at[:NNODES, 0].set(batch)
    zrows = jnp.zeros((NP, 128), jnp.float32)

    deg = _deg_call(dst)
    deg2 = deg.reshape(NP, 1)

    h1 = _mm1_call(x, W1, deg2)
    a1 = _agg_col_call(src, dst, h1)
    s1 = _stats_call(a1, deg2, b1.reshape(1, 256), 256, "col")
    h2 = _mm_mid_call(a1, deg2, b1.reshape(1, 256), g1.reshape(1, 256),
                      be1.reshape(1, 256), s1, W2, 256, "col")
    a2 = _agg_edge_call(src, dst, h2, zrows)
    s2 = _stats_call(a2, deg2, b2.reshape(1, 128), 128, "sum")
    h3 = _mm_mid_call(a2, deg2, b2.reshape(1, 128), g2.reshape(1, 128),
                      be2.reshape(1, 128), s2, W3, 128, "sum")
    a3 = _agg_edge_call(src, dst, h3, zrows)
    s3 = _stats_call(a3, deg2, b3.reshape(1, 128), 128, "sum")
    p = _pool_call(a3, deg2, b3.reshape(1, 128), g3.reshape(1, 128),
                   be3.reshape(1, 128), s3, batch2d)

    fw3p = jnp.zeros((64, 8), jnp.float32).at[:, :1].set(fW3)
    fb3p = jnp.zeros((1, 8), jnp.float32).at[0, 0].set(fb3[0])
    out = _head_call(p, fW1, fb1.reshape(1, 64), g4.reshape(1, 64),
                     be4.reshape(1, 64), fW2, fb2.reshape(1, 64), fw3p, fb3p)
    return out[:, 0]


# speedup vs baseline: 17.5466x; 1.3894x over previous
import functools

import jax
import jax.numpy as jnp
from jax import lax
from jax.experimental import pallas as pl
from jax.experimental.pallas import tpu as pltpu
from jax.experimental.pallas import tpu_sc as plsc

NNODES = 10000
NP = 10240
EDGES = 160000
NGRAPH = 64
RB = 2048
NRB = NP // RB
CH = 128
TILES = 16
RPT = NP // TILES
NCH = EDGES // CH
GC = 10
GE = GC * CH
NGRP = EDGES // GE
EPSBN = 1e-5
FNN = float(NNODES)


def _sc_mesh():
    return plsc.VectorSubcoreMesh(core_axis_name="c", subcore_axis_name="s",
                                  num_cores=2, num_subcores=TILES)


def _deg_call(ei):
    @functools.partial(
        pl.kernel,
        mesh=_sc_mesh(),
        out_type=jax.ShapeDtypeStruct((NP,), jnp.float32),
        scratch_types=[
            pltpu.VMEM_SHARED((NP,), jnp.float32),
            pltpu.VMEM((RPT,), jnp.float32),
            pltpu.VMEM((GE,), jnp.int32),
            pltpu.VMEM((CH,), jnp.int32),
            pltpu.VMEM((CH,), jnp.int32),
            pltpu.VMEM((CH,), jnp.float32),
            pltpu.SemaphoreType.DMA,
        ],
    )
    def k(ei_hbm, deg_out, acc, ones_r, dstg_v, dsta_v, dstb_v, ones_c, sem):
        c = lax.axis_index("c")
        s = lax.axis_index("s")
        for i in range(RPT // 16):
            ones_r[pl.ds(i * 16, 16)] = jnp.ones((16,), jnp.float32)
        for i in range(CH // 16):
            ones_c[pl.ds(i * 16, 16)] = jnp.ones((16,), jnp.float32)
        r0 = s * RPT
        dstb = [dsta_v, dstb_v]

        @pl.when(c == 0)
        def _():
            pltpu.sync_copy(ones_r, acc.at[pl.ds(r0, RPT)])

        plsc.subcore_barrier()

        @pl.when(c == 0)
        def _():
            @pl.loop(s, NGRP, step=TILES)
            def _(g):
                pltpu.sync_copy(ei_hbm.at[1, pl.ds(g * GE, GE)], dstg_v)
                descs = [None] * GC
                for j in range(GC):
                    nb = j % 2
                    if j >= 2:
                        descs[j - 2].wait()
                    for i in range(CH // 16):
                        sl = pl.ds(i * 16, 16)
                        dstb[nb][sl] = dstg_v[pl.ds(j * CH + i * 16, 16)]
                    descs[j] = pltpu.async_copy(ones_c, acc.at[dstb[nb]],
                                                sem, add=True)
                descs[GC - 2].wait()
                descs[GC - 1].wait()

        plsc.subcore_barrier()

        @pl.when(c == 0)
        def _():
            pltpu.sync_copy(acc.at[pl.ds(r0, RPT)], deg_out.at[pl.ds(r0, RPT)])

    return k(ei)


def _agg_col_call(ei, h):
    @functools.partial(
        pl.kernel,
        mesh=_sc_mesh(),
        out_type=jax.ShapeDtypeStruct((2 * NP, 128), jnp.float32),
        scratch_types=[
            pltpu.VMEM_SHARED((NP, 128), jnp.float32),
            pltpu.VMEM((GE,), jnp.int32),
            pltpu.VMEM((GE,), jnp.int32),
            pltpu.VMEM((CH,), jnp.int32),
            pltpu.VMEM((CH,), jnp.int32),
            pltpu.VMEM((CH,), jnp.int32),
            pltpu.VMEM((CH, 128), jnp.float32),
            pltpu.VMEM((CH, 128), jnp.float32),
            pltpu.SemaphoreType.DMA,
        ],
    )
    def k(ei_hbm, h_hbm, out_hbm, acc, srcg_v, dstg_v, sca_v, scb_v,
          dstc_v, rowa_v, rowb_v, sem):
        c = lax.axis_index("c")
        s = lax.axis_index("s")
        base = c * NP
        r0 = s * RPT
        pltpu.sync_copy(h_hbm.at[pl.ds(base + r0, RPT)], acc.at[pl.ds(r0, RPT)])
        plsc.subcore_barrier()
        srcb = [sca_v, scb_v]
        rowb = [rowa_v, rowb_v]

        @pl.loop(s, NGRP, step=TILES)
        def _(g):
            e0 = g * GE
            pltpu.sync_copy(ei_hbm.at[0, pl.ds(e0, GE)], srcg_v)
            pltpu.sync_copy(ei_hbm.at[1, pl.ds(e0, GE)], dstg_v)
            for i in range(CH // 16):
                sl = pl.ds(i * 16, 16)
                srcb[0][sl] = srcg_v[sl] + base
            cp = pltpu.async_copy(h_hbm.at[srcb[0]], rowb[0], sem)
            for j in range(GC):
                if j + 1 < GC:
                    nb = (j + 1) % 2
                    for i in range(CH // 16):
                        sl = pl.ds(i * 16, 16)
                        srcb[nb][sl] = srcg_v[pl.ds((j + 1) * CH + i * 16, 16)] + base
                    cpn = pltpu.async_copy(h_hbm.at[srcb[nb]], rowb[nb], sem)
                for i in range(CH // 16):
                    sl = pl.ds(i * 16, 16)
                    dstc_v[sl] = dstg_v[pl.ds(j * CH + i * 16, 16)]
                cp.wait()
                pltpu.sync_copy(rowb[j % 2], acc.at[dstc_v], add=True)
                if j + 1 < GC:
                    cp = cpn

        plsc.subcore_barrier()
        pltpu.sync_copy(acc.at[pl.ds(r0, RPT)],
                        out_hbm.at[pl.ds(base + r0, RPT)])

    return k(ei, h)


def _agg_edge_call(ei, h, zrows):
    @functools.partial(
        pl.kernel,
        mesh=_sc_mesh(),
        out_type=jax.ShapeDtypeStruct((2 * NP, 128), jnp.float32),
        scratch_types=[
            pltpu.VMEM_SHARED((NP, 128), jnp.float32),
            pltpu.VMEM((GE,), jnp.int32),
            pltpu.VMEM((GE,), jnp.int32),
            pltpu.VMEM((CH,), jnp.int32),
            pltpu.VMEM((CH, 128), jnp.float32),
            pltpu.VMEM((CH, 128), jnp.float32),
            pltpu.SemaphoreType.DMA,
        ],
    )
    def k(ei_hbm, h_hbm, z_hbm, out_hbm, acc, srcg_v, dstg_v,
          dstc_v, rowa_v, rowb_v, sem):
        c = lax.axis_index("c")
        s = lax.axis_index("s")
        r0 = s * RPT

        @pl.when(c == 0)
        def _():
            pltpu.sync_copy(h_hbm.at[pl.ds(r0, RPT)], acc.at[pl.ds(r0, RPT)])

        @pl.when(c == 1)
        def _():
            pltpu.sync_copy(z_hbm.at[pl.ds(r0, RPT)], acc.at[pl.ds(r0, RPT)])

        plsc.subcore_barrier()
        rowb = [rowa_v, rowb_v]
        half = NGRP // 2

        @pl.loop(c * half + s, half + c * (NGRP - half), step=TILES)
        def _(g):
            e0 = g * GE
            pltpu.sync_copy(ei_hbm.at[0, pl.ds(e0, GE)], srcg_v)
            pltpu.sync_copy(ei_hbm.at[1, pl.ds(e0, GE)], dstg_v)
            cp = pltpu.async_copy(h_hbm.at[srcg_v.at[pl.ds(0, CH)]], rowb[0], sem)
            for j in range(GC):
                if j + 1 < GC:
                    cpn = pltpu.async_copy(
                        h_hbm.at[srcg_v.at[pl.ds((j + 1) * CH, CH)]],
                        rowb[(j + 1) % 2], sem)
                for i in range(CH // 16):
                    sl = pl.ds(i * 16, 16)
                    dstc_v[sl] = dstg_v[pl.ds(j * CH + i * 16, 16)]
                cp.wait()
                pltpu.sync_copy(rowb[j % 2], acc.at[dstc_v], add=True)
                if j + 1 < GC:
                    cp = cpn

        plsc.subcore_barrier()
        pltpu.sync_copy(acc.at[pl.ds(r0, RPT)],
                        out_hbm.at[pl.ds(c * NP + r0, RPT)])

    return k(ei, h, zrows)


def _mm1_call(xp, w1p, deg2):
    def body(x_ref, w_ref, deg_ref, out_ref):
        i = pl.program_id(0)
        dis = lax.rsqrt(deg_ref[...])
        h = jnp.dot(x_ref[...].astype(jnp.bfloat16), w_ref[...].astype(jnp.bfloat16),
                    preferred_element_type=jnp.float32)
        ridx = i * RB + lax.broadcasted_iota(jnp.int32, (RB, 1), 0)
        out_ref[...] = jnp.where(ridx < NNODES, h * dis, 0.0)

    return pl.pallas_call(
        body,
        grid=(NRB, 2),
        in_specs=[
            pl.BlockSpec((RB, 373), lambda i, j: (i, 0)),
            pl.BlockSpec((373, 128), lambda i, j: (0, j)),
            pl.BlockSpec((RB, 1), lambda i, j: (i, 0)),
        ],
        out_specs=pl.BlockSpec((RB, 128), lambda i, j: (j * NRB + i, 0)),
        out_shape=jax.ShapeDtypeStruct((2 * NP, 128), jnp.float32),
    )(xp, w1p, deg2)


def _relu_rows(a0, a1, dis, b, mode, f2):
    if mode == "col":
        y0 = jnp.maximum(a0 * dis + b[:, :f2], 0.0)
        y1 = jnp.maximum(a1 * dis + b[:, f2:], 0.0)
        return y0, y1
    y = jnp.maximum((a0 + a1) * dis + b, 0.0)
    return y, None


def _bn_affine(st, g, be):
    mu = st[0:1, :] * (1.0 / FNN)
    var = st[1:2, :] * (1.0 / FNN) - mu * mu
    alpha = g * lax.rsqrt(var + EPSBN)
    beta = be - mu * alpha
    return alpha, beta


def _stats_call(a, deg2, b2d, f, mode):
    f2 = f // 2 if mode == "col" else f

    def body(a0_ref, a1_ref, deg_ref, b_ref, out_ref):
        i = pl.program_id(0)

        @pl.when(i == 0)
        def _():
            out_ref[...] = jnp.zeros((8, f), jnp.float32)

        dis = lax.rsqrt(deg_ref[...])
        ridx = i * RB + lax.broadcasted_iota(jnp.int32, (RB, 1), 0)
        mask = (ridx < NNODES).astype(jnp.float32)
        y0, y1 = _relu_rows(a0_ref[...], a1_ref[...], dis, b_ref[...], mode, f2)
        if mode == "col":
            y0 = y0 * mask
            y1 = y1 * mask
            out_ref[0:1, :f2] += jnp.sum(y0, axis=0, keepdims=True)
            out_ref[0:1, f2:] += jnp.sum(y1, axis=0, keepdims=True)
            out_ref[1:2, :f2] += jnp.sum(y0 * y0, axis=0, keepdims=True)
            out_ref[1:2, f2:] += jnp.sum(y1 * y1, axis=0, keepdims=True)
        else:
            y0 = y0 * mask
            out_ref[0:1, :] += jnp.sum(y0, axis=0, keepdims=True)
            out_ref[1:2, :] += jnp.sum(y0 * y0, axis=0, keepdims=True)

    return pl.pallas_call(
        body,
        grid=(NRB,),
        in_specs=[
            pl.BlockSpec((RB, f2), lambda i: (i, 0)),
            pl.BlockSpec((RB, f2), lambda i: (NRB + i, 0)),
            pl.BlockSpec((RB, 1), lambda i: (i, 0)),
            pl.BlockSpec((1, f), lambda i: (0, 0)),
        ],
        out_specs=pl.BlockSpec((8, f), lambda i: (0, 0)),
        out_shape=jax.ShapeDtypeStruct((8, f), jnp.float32),
    )(a, a, deg2, b2d)


def _mm_mid_call(a, deg2, b2d, g2d, be2d, stats, w, fin, mode):
    f2 = fin // 2 if mode == "col" else fin

    def body(a0_ref, a1_ref, deg_ref, b_ref, g_ref, be_ref, st_ref, w_ref,
             out_ref):
        dis = lax.rsqrt(deg_ref[...])
        alpha, beta = _bn_affine(st_ref[...], g_ref[...], be_ref[...])
        y0, y1 = _relu_rows(a0_ref[...], a1_ref[...], dis, b_ref[...], mode, f2)
        w_blk = w_ref[...]
        if mode == "col":
            yn0 = y0 * alpha[:, :f2] + beta[:, :f2]
            yn1 = y1 * alpha[:, f2:] + beta[:, f2:]
            wb = w_blk.astype(jnp.bfloat16)
            h = (jnp.dot(yn0.astype(jnp.bfloat16), wb[:f2, :],
                         preferred_element_type=jnp.float32)
                 + jnp.dot(yn1.astype(jnp.bfloat16), wb[f2:, :],
                           preferred_element_type=jnp.float32))
        else:
            yn = y0 * alpha + beta
            h = jnp.dot(yn.astype(jnp.bfloat16), w_blk.astype(jnp.bfloat16),
                        preferred_element_type=jnp.float32)
        out_ref[...] = h * dis

    return pl.pallas_call(
        body,
        grid=(NRB,),
        in_specs=[
            pl.BlockSpec((RB, f2), lambda i: (i, 0)),
            pl.BlockSpec((RB, f2), lambda i: (NRB + i, 0)),
            pl.BlockSpec((RB, 1), lambda i: (i, 0)),
            pl.BlockSpec((1, fin), lambda i: (0, 0)),
            pl.BlockSpec((1, fin), lambda i: (0, 0)),
            pl.BlockSpec((1, fin), lambda i: (0, 0)),
            pl.BlockSpec((8, fin), lambda i: (0, 0)),
            pl.BlockSpec((fin, 128), lambda i: (0, 0)),
        ],
        out_specs=pl.BlockSpec((RB, 128), lambda i: (i, 0)),
        out_shape=jax.ShapeDtypeStruct((NP, 128), jnp.float32),
    )(a, a, deg2, b2d, g2d, be2d, stats, w)


def _pool_call(a, deg2, b2d, g2d, be2d, stats, batch2d):
    def body(a0_ref, a1_ref, deg_ref, b_ref, g_ref, be_ref, st_ref, bat_ref,
             out_ref):
        i = pl.program_id(0)

        @pl.when(i == 0)
        def _():
            out_ref[...] = jnp.zeros((NGRAPH, 128), jnp.float32)

        dis = lax.rsqrt(deg_ref[...])
        alpha, beta = _bn_affine(st_ref[...], g_ref[...], be_ref[...])
        y, _ = _relu_rows(a0_ref[...], a1_ref[...], dis, b_ref[...], "sum", 128)
        yn = y * alpha + beta
        gid = lax.broadcasted_iota(jnp.int32, (RB, NGRAPH), 1)
        onehot = (bat_ref[...] == gid).astype(jnp.float32)
        dn = (((0,), (0,)), ((), ()))
        out_ref[...] += lax.dot_general(onehot, yn, dn,
                                        preferred_element_type=jnp.float32,
                                        precision=lax.Precision.HIGHEST)

    return pl.pallas_call(
        body,
        grid=(NRB,),
        in_specs=[
            pl.BlockSpec((RB, 128), lambda i: (i, 0)),
            pl.BlockSpec((RB, 128), lambda i: (NRB + i, 0)),
            pl.BlockSpec((RB, 1), lambda i: (i, 0)),
            pl.BlockSpec((1, 128), lambda i: (0, 0)),
            pl.BlockSpec((1, 128), lambda i: (0, 0)),
            pl.BlockSpec((1, 128), lambda i: (0, 0)),
            pl.BlockSpec((8, 128), lambda i: (0, 0)),
            pl.BlockSpec((RB, 1), lambda i: (i, 0)),
        ],
        out_specs=pl.BlockSpec((NGRAPH, 128), lambda i: (0, 0)),
        out_shape=jax.ShapeDtypeStruct((NGRAPH, 128), jnp.float32),
    )(a, a, deg2, b2d, g2d, be2d, stats, batch2d)


def _head_call(p, fw1, fb1, g4, be4, fw2, fb2, fw3p, fb3p):
    def body(p_ref, w1_ref, b1_ref, g_ref, be_ref, w2_ref, b2_ref, w3_ref,
             b3_ref, out_ref):
        q = jnp.maximum(
            jnp.dot(p_ref[...].astype(jnp.bfloat16), w1_ref[...].astype(jnp.bfloat16),
                    preferred_element_type=jnp.float32) + b1_ref[...], 0.0)
        mu = jnp.mean(q, axis=0, keepdims=True)
        var = jnp.mean(q * q, axis=0, keepdims=True) - mu * mu
        qn = (q - mu) * lax.rsqrt(var + EPSBN) * g_ref[...] + be_ref[...]
        r = jnp.maximum(
            jnp.dot(qn.astype(jnp.bfloat16), w2_ref[...].astype(jnp.bfloat16),
                    preferred_element_type=jnp.float32) + b2_ref[...], 0.0)
        t = jnp.dot(r.astype(jnp.bfloat16), w3_ref[...].astype(jnp.bfloat16),
                    preferred_element_type=jnp.float32) + b3_ref[...]
        out_ref[...] = jnp.maximum(t, 0.0)

    return pl.pallas_call(
        body,
        out_shape=jax.ShapeDtypeStruct((NGRAPH, 8), jnp.float32),
    )(p, fw1, fb1, g4, be4, fw2, fb2, fw3p, fb3p)


def kernel(x, edge_index, batch, W1, b1, g1, be1, W2, b2, g2, be2, W3, b3,
           g3, be3, fW1, fb1, g4, be4, fW2, fb2, fW3, fb3):
    batch2d = jnp.full((NP, 1), NGRAPH, jnp.int32).at[:NNODES, 0].set(batch)
    zrows = jnp.zeros((NP, 128), jnp.float32)

    deg = _deg_call(edge_index)
    deg2 = deg.reshape(NP, 1)

    h1 = _mm1_call(x, W1, deg2)
    a1 = _agg_col_call(edge_index, h1)
    s1 = _stats_call(a1, deg2, b1.reshape(1, 256), 256, "col")
    h2 = _mm_mid_call(a1, deg2, b1.reshape(1, 256), g1.reshape(1, 256),
                      be1.reshape(1, 256), s1, W2, 256, "col")
    a2 = _agg_edge_call(edge_index, h2, zrows)
    s2 = _stats_call(a2, deg2, b2.reshape(1, 128), 128, "sum")
    h3 = _mm_mid_call(a2, deg2, b2.reshape(1, 128), g2.reshape(1, 128),
                      be2.reshape(1, 128), s2, W3, 128, "sum")
    a3 = _agg_edge_call(edge_index, h3, zrows)
    s3 = _stats_call(a3, deg2, b3.reshape(1, 128), 128, "sum")
    p = _pool_call(a3, deg2, b3.reshape(1, 128), g3.reshape(1, 128),
                   be3.reshape(1, 128), s3, batch2d)

    fw3p = jnp.zeros((64, 8), jnp.float32).at[:, :1].set(fW3)
    fb3p = jnp.zeros((1, 8), jnp.float32).at[0, 0].set(fb3[0])
    out = _head_call(p, fW1, fb1.reshape(1, 64), g4.reshape(1, 64),
                     be4.reshape(1, 64), fW2, fb2.reshape(1, 64), fw3p, fb3p)
    return out[:, 0]
